# baseline (speedup 1.0000x reference)
import functools

import jax
import jax.numpy as jnp
from jax import lax
from jax.experimental import pallas as pl
from jax.experimental.pallas import tpu as pltpu
from jax.experimental.pallas import tpu_sc as plsc

_CH = 32


def kernel(x, pe_table):
    B, S, D = x.shape
    info = plsc.get_sparse_core_info()
    nc, ns = info.num_cores, info.num_subcores
    nw = nc * ns
    rows = B * S
    rows_per_w = rows // nw
    n_chunks = rows_per_w // _CH
    workers_per_batch = nw // B

    mesh = plsc.VectorSubcoreMesh(core_axis_name="c", subcore_axis_name="s")

    @functools.partial(
        pl.kernel,
        mesh=mesh,
        out_type=jax.ShapeDtypeStruct((rows, D), jnp.float32),
        scratch_types=[
            pltpu.VMEM((_CH, D), jnp.float32),
            pltpu.VMEM((_CH, D), jnp.float32),
        ],
    )
    def sc_add(x_hbm, pe_hbm, out_hbm, x_v, pe_v):
        wid = lax.axis_index("s") * nc + lax.axis_index("c")
        row_base = wid * rows_per_w
        pe_base = (wid % workers_per_batch) * rows_per_w

        def chunk_body(ci, _):
            r0 = row_base + ci * _CH
            p0 = pe_base + ci * _CH
            pltpu.sync_copy(x_hbm.at[pl.ds(r0, _CH)], x_v)
            pltpu.sync_copy(pe_hbm.at[pl.ds(p0, _CH)], pe_v)

            def row_body(r, _):
                def col_body(c, _):
                    sl = pl.ds(c * 16, 16)
                    x_v[r, sl] = x_v[r, sl] + pe_v[r, sl]
                    return 0

                return lax.fori_loop(0, D // 16, col_body, 0)

            lax.fori_loop(0, _CH, row_body, 0)
            pltpu.sync_copy(x_v, out_hbm.at[pl.ds(r0, _CH)])
            return 0

        lax.fori_loop(0, n_chunks, chunk_body, 0)

    out2d = sc_add(x.reshape(rows, D), pe_table)
    return out2d.reshape(B, S, D)


# speedup vs baseline: 5.4421x; 5.4421x over previous
import jax
import jax.numpy as jnp
from jax.experimental import pallas as pl
from jax.experimental.pallas import tpu as pltpu

_S_BLK = 2048


def _add_pe_kernel(x_ref, pe_ref, o_ref):
    o_ref[...] = x_ref[...] + pe_ref[...][None, :, :]


def kernel(x, pe_table):
    B, S, D = x.shape
    n_s = S // _S_BLK
    return pl.pallas_call(
        _add_pe_kernel,
        grid=(n_s, B),
        in_specs=[
            pl.BlockSpec((1, _S_BLK, D), lambda i, b: (b, i, 0)),
            pl.BlockSpec((_S_BLK, D), lambda i, b: (i, 0)),
        ],
        out_specs=pl.BlockSpec((1, _S_BLK, D), lambda i, b: (b, i, 0)),
        out_shape=jax.ShapeDtypeStruct((B, S, D), x.dtype),
        compiler_params=pltpu.CompilerParams(
            dimension_semantics=("parallel", "parallel"),
        ),
    )(x, pe_table)
